# baseline (speedup 1.0000x reference)
import functools

import jax
import jax.numpy as jnp
from jax import lax
from jax.experimental import pallas as pl
from jax.experimental.pallas import tpu as pltpu
from jax.experimental.pallas import tpu_sc as plsc

L = 16
B = 16384
K = 16
NW = 32
BPW = B // NW
NG = BPW // L

_mesh = plsc.VectorSubcoreMesh(
    core_axis_name="c", subcore_axis_name="s", num_cores=2, num_subcores=16
)


@functools.partial(
    pl.kernel,
    out_type=jax.ShapeDtypeStruct((B,), jnp.float32),
    mesh=_mesh,
    scratch_types=[
        pltpu.VMEM((BPW,), jnp.int32),
        pltpu.VMEM((BPW,), jnp.int32),
        pltpu.VMEM((K * BPW,), jnp.int32),
        pltpu.VMEM((K * BPW,), jnp.int32),
        pltpu.VMEM((K * BPW,), jnp.float32),
        pltpu.VMEM((K * BPW,), jnp.float32),
        pltpu.VMEM((BPW,), jnp.float32),
        pltpu.VMEM((BPW,), jnp.float32),
        pltpu.VMEM((BPW,), jnp.float32),
        pltpu.VMEM((L,), jnp.float32),
        pltpu.SemaphoreType.DMA,
    ],
    compiler_params=pltpu.CompilerParams(needs_layout_passes=False),
)
def _fm(u_hbm, m_hbm, w_hbm, v1_hbm, w0_hbm, out_hbm,
        uv, mv, iu, im, vu, vm, wu, wm, ov, w0v, sem):
    wid = lax.axis_index("s") * 2 + lax.axis_index("c")
    base = wid * BPW
    pltpu.sync_copy(u_hbm.at[pl.ds(base, BPW)], uv)
    pltpu.sync_copy(m_hbm.at[pl.ds(base, BPW)], mv)

    def build(g, c):
        s = g * L
        u16 = uv[pl.ds(s, L)] * K
        m16 = mv[pl.ds(s, L)] * K
        for kk in range(K):
            iu[pl.ds(kk * BPW + s, L)] = u16 + kk
            im[pl.ds(kk * BPW + s, L)] = m16 + kk
        return c

    lax.fori_loop(0, NG, build, 0)

    c1 = pltpu.async_copy(v1_hbm.at[iu], vu, sem)
    c2 = pltpu.async_copy(v1_hbm.at[im], vm, sem)
    c3 = pltpu.async_copy(w_hbm.at[uv], wu, sem)
    c4 = pltpu.async_copy(w_hbm.at[mv], wm, sem)
    c1.wait()
    c2.wait()
    c3.wait()
    c4.wait()
    pltpu.sync_copy(w0_hbm, w0v)
    w0vec = w0v[...]

    def body(g, c):
        s = g * L
        acc = w0vec + wu[pl.ds(s, L)] + wm[pl.ds(s, L)]
        for kk in range(K):
            acc = acc + vu[pl.ds(kk * BPW + s, L)] * vm[pl.ds(kk * BPW + s, L)]
        ov[pl.ds(s, L)] = acc
        return c

    lax.fori_loop(0, NG, body, 0)
    pltpu.sync_copy(ov, out_hbm.at[pl.ds(base, BPW)])


def kernel(idx, w0, w, V):
    idx = idx.astype(jnp.int32)
    u = idx[:, 0]
    m = idx[:, 1]
    v1 = V.reshape(-1)
    w0b = jnp.broadcast_to(w0.astype(jnp.float32), (L,))
    return _fm(u, m, w, v1, w0b)


# speedup vs baseline: 1.0304x; 1.0304x over previous
import functools

import jax
import jax.numpy as jnp
from jax import lax
from jax.experimental import pallas as pl
from jax.experimental.pallas import tpu as pltpu
from jax.experimental.pallas import tpu_sc as plsc

L = 16
B = 16384
K = 16
NW = 32
BPW = B // NW
QS = 128
NQ = BPW // QS
GPQ = QS // L
SR = 128 // K

_mesh = plsc.VectorSubcoreMesh(
    core_axis_name="c", subcore_axis_name="s", num_cores=2, num_subcores=16
)


@functools.partial(
    pl.kernel,
    out_type=jax.ShapeDtypeStruct((B,), jnp.float32),
    mesh=_mesh,
    scratch_types=[
        pltpu.VMEM((BPW,), jnp.int32),
        pltpu.VMEM((BPW,), jnp.int32),
        pltpu.VMEM((QS,), jnp.int32),
        pltpu.VMEM((QS,), jnp.int32),
        pltpu.VMEM((QS,), jnp.int32),
        pltpu.VMEM((QS,), jnp.int32),
        pltpu.VMEM((QS,), jnp.int32),
        pltpu.VMEM((QS,), jnp.int32),
        pltpu.VMEM((QS,), jnp.int32),
        pltpu.VMEM((QS,), jnp.int32),
        pltpu.VMEM((QS, 128), jnp.float32),
        pltpu.VMEM((QS, 128), jnp.float32),
        pltpu.VMEM((QS, 128), jnp.float32),
        pltpu.VMEM((QS, 128), jnp.float32),
        pltpu.VMEM((BPW,), jnp.float32),
        pltpu.VMEM((BPW,), jnp.float32),
        pltpu.VMEM((BPW,), jnp.float32),
        pltpu.VMEM((L,), jnp.float32),
        pltpu.SemaphoreType.DMA,
        pltpu.SemaphoreType.DMA,
    ],
    compiler_params=pltpu.CompilerParams(needs_layout_passes=False),
)
def _fm(u_hbm, m_hbm, w_hbm, v128_hbm, w0_hbm, out_hbm,
        uv, mv, ru0, ru1, ru2, ru3, rm0, rm1, rm2, rm3,
        vux, vmx, vuy, vmy, wu, wm, ov, w0v, semx, semy):
    wid = lax.axis_index("s") * 2 + lax.axis_index("c")
    base = wid * BPW
    pltpu.sync_copy(u_hbm.at[pl.ds(base, BPW)], uv)
    pltpu.sync_copy(m_hbm.at[pl.ds(base, BPW)], mv)
    cwu = pltpu.async_copy(w_hbm.at[uv], wu, semy)
    cwm = pltpu.async_copy(w_hbm.at[mv], wm, semy)

    rus = (ru0, ru1, ru2, ru3)
    rms = (rm0, rm1, rm2, rm3)
    for q in range(NQ):

        def build(g, c, q=q):
            s = q * QS + g * L
            rus[q][pl.ds(g * L, L)] = lax.shift_right_logical(uv[pl.ds(s, L)], 3)
            rms[q][pl.ds(g * L, L)] = lax.shift_right_logical(mv[pl.ds(s, L)], 3)
            return c

        lax.fori_loop(0, GPQ, build, 0)

    pltpu.sync_copy(w0_hbm, w0v)
    w0vec = w0v[...]
    lanes = lax.iota(jnp.int32, L)

    bufs = ((vux, vmx, semx), (vuy, vmy, semy))
    copies = [None] * NQ

    def fire(q):
        vu_b, vm_b, sem = bufs[q % 2]
        copies[q] = (
            pltpu.async_copy(v128_hbm.at[rus[q]], vu_b, sem),
            pltpu.async_copy(v128_hbm.at[rms[q]], vm_b, sem),
        )

    def compute(q):
        vu_b, vm_b, _ = bufs[q % 2]

        def body(g, c, q=q, vu_b=vu_b, vm_b=vm_b):
            s = q * QS + g * L
            rows = g * L + lanes
            uvec = uv[pl.ds(s, L)]
            mvec = mv[pl.ds(s, L)]
            cu = (uvec & 7) * K
            cm = (mvec & 7) * K
            acc = w0vec + wu[pl.ds(s, L)] + wm[pl.ds(s, L)]
            for kk in range(K):
                a = plsc.load_gather(vu_b, [rows, cu + kk])
                b = plsc.load_gather(vm_b, [rows, cm + kk])
                acc = acc + a * b
            ov[pl.ds(s, L)] = acc
            return c

        lax.fori_loop(0, GPQ, body, 0)

    fire(0)
    fire(1)
    cwu.wait()
    cwm.wait()
    for q in range(NQ):
        copies[q][0].wait()
        copies[q][1].wait()
        compute(q)
        if q + 2 < NQ:
            fire(q + 2)

    pltpu.sync_copy(ov, out_hbm.at[pl.ds(base, BPW)])


def kernel(idx, w0, w, V):
    idx = idx.astype(jnp.int32)
    u = idx[:, 0]
    m = idx[:, 1]
    v128 = V.reshape(-1, 128)
    w0b = jnp.broadcast_to(w0.astype(jnp.float32), (L,))
    return _fm(u, m, w, v128, w0b)
